# baseline (speedup 1.0000x reference)
import functools

import jax
import jax.numpy as jnp
from jax import lax
from jax.experimental import pallas as pl
from jax.experimental.pallas import tpu as pltpu
from jax.experimental.pallas import tpu_sc as plsc

LATENT_DIM = 32
BATCH = 16384

_info = plsc.get_sparse_core_info()
_NC, _NS = _info.num_cores, _info.num_subcores
_NW = _NC * _NS
_B_PER_W = BATCH // _NW

_mesh = plsc.VectorSubcoreMesh(core_axis_name="c", subcore_axis_name="s")


@functools.partial(
    pl.kernel,
    mesh=_mesh,
    out_type=(
        jax.ShapeDtypeStruct((BATCH, LATENT_DIM), jnp.float32),
        jax.ShapeDtypeStruct((BATCH, LATENT_DIM), jnp.float32),
    ),
    scratch_types=[
        pltpu.VMEM((_B_PER_W,), jnp.int32),
        pltpu.VMEM((_B_PER_W,), jnp.int32),
        pltpu.VMEM((_B_PER_W, LATENT_DIM), jnp.float32),
        pltpu.VMEM((_B_PER_W, LATENT_DIM), jnp.float32),
        pltpu.SemaphoreType.DMA,
        pltpu.SemaphoreType.DMA,
    ],
    compiler_params=pltpu.CompilerParams(use_tc_tiling_on_sc=False),
)
def _dual_gather(uids_hbm, iids_hbm, utab_hbm, itab_hbm, uout_hbm, iout_hbm,
                 uids_v, iids_v, urows_v, irows_v, usem, isem):
    wid = lax.axis_index("s") * _NC + lax.axis_index("c")
    base = wid * _B_PER_W
    pltpu.sync_copy(uids_hbm.at[pl.ds(base, _B_PER_W)], uids_v)
    pltpu.sync_copy(iids_hbm.at[pl.ds(base, _B_PER_W)], iids_v)
    uc = pltpu.async_copy(utab_hbm.at[uids_v], urows_v, usem)
    ic = pltpu.async_copy(itab_hbm.at[iids_v], irows_v, isem)
    uc.wait()
    pltpu.sync_copy(urows_v, uout_hbm.at[pl.ds(base, _B_PER_W)])
    ic.wait()
    pltpu.sync_copy(irows_v, iout_hbm.at[pl.ds(base, _B_PER_W)])


def kernel(user_ids, item_ids, user_table, item_table):
    return _dual_gather(user_ids, item_ids, user_table, item_table)
